# baseline (speedup 1.0000x reference)
import functools

import jax
import jax.numpy as jnp
from jax import lax
from jax.experimental import pallas as pl
from jax.experimental.pallas import tpu as pltpu
from jax.experimental.pallas import tpu_sc as plsc

_N_NODES = 10000
_D = 128
_N_GRAPHS = 512
_N_CONV = 3

_BLK = 1000
_NB = _N_NODES // _BLK

_NC = 2
_NS = 16
_CH = 128
_C = 157
_EPAD = _NS * _C * _CH
_HN = 5120
_NPAD = 5376
_RPT = _NPAD // _NS
_HN1 = _N_NODES - _HN


def _softplus(z):
    return jnp.maximum(z, 0.0) + jnp.log1p(jnp.exp(-jnp.abs(z)))


def _sigmoid(z):
    return 1.0 / (1.0 + jnp.exp(-z))


def _dot_t(a, b):
    return lax.dot_general(a, b, (((1,), (1,)), ((), ())),
                           preferred_element_type=jnp.float32)


def _embed_msg_body(nodes_ref, embw_ref, w1_ref, w2_ref, x_ref, m_ref):
    xb = _dot_t(nodes_ref[...], embw_ref[...])
    z1 = _dot_t(xb, w1_ref[...])
    z2 = _dot_t(xb, w2_ref[...])
    x_ref[...] = xb
    m_ref[...] = _sigmoid(z1) * _softplus(z2)


def _update_msg_body(x_ref, agg_ref, w1_ref, w2_ref, xn_ref, m_ref):
    xn = x_ref[...] + _softplus(agg_ref[...])
    z1 = _dot_t(xn, w1_ref[...])
    z2 = _dot_t(xn, w2_ref[...])
    xn_ref[...] = xn
    m_ref[...] = _sigmoid(z1) * _softplus(z2)


def _final_body(x_ref, agg_ref, gidx_ref, invc_ref, fcw_ref, fcb_ref,
                regw_ref, regb_ref, out_ref, acc_ref):
    i = pl.program_id(0)

    @pl.when(i == 0)
    def _():
        acc_ref[...] = jnp.zeros_like(acc_ref)

    xb = x_ref[...] + _softplus(agg_ref[...])
    g = gidx_ref[0]
    iota = lax.broadcasted_iota(jnp.int32, (_N_GRAPHS, _BLK), 0)
    onehot = (iota == g).astype(jnp.float32)
    acc_ref[...] += lax.dot_general(onehot, xb, (((1,), (0,)), ((), ())),
                                    preferred_element_type=jnp.float32)

    @pl.when(i == _NB - 1)
    def _():
        pooled = acc_ref[...] * invc_ref[...]
        y = _softplus(_dot_t(pooled, fcw_ref[0]) + fcb_ref[0])
        y = _softplus(_dot_t(y, fcw_ref[1]) + fcb_ref[1])
        out_ref[...] = (jnp.sum(y * regw_ref[...], axis=1, keepdims=True)
                        + regb_ref[...])


def _tc_embed_msg(nodes, emb_w, w1, w2):
    full = lambda i: (0, 0)
    blk = lambda i: (i, 0)
    return pl.pallas_call(
        _embed_msg_body,
        grid=(_NB,),
        in_specs=[
            pl.BlockSpec((_BLK, _D), blk),
            pl.BlockSpec((_D, _D), full),
            pl.BlockSpec((_D, _D), full),
            pl.BlockSpec((_D, _D), full),
        ],
        out_specs=[
            pl.BlockSpec((_BLK, _D), blk),
            pl.BlockSpec((_BLK, _D), blk),
        ],
        out_shape=[
            jax.ShapeDtypeStruct((_N_NODES, _D), jnp.float32),
            jax.ShapeDtypeStruct((_N_NODES, _D), jnp.float32),
        ],
    )(nodes, emb_w, w1, w2)


def _tc_update_msg(x, agg, w1, w2):
    full = lambda i: (0, 0)
    blk = lambda i: (i, 0)
    return pl.pallas_call(
        _update_msg_body,
        grid=(_NB,),
        in_specs=[
            pl.BlockSpec((_BLK, _D), blk),
            pl.BlockSpec((_BLK, _D), blk),
            pl.BlockSpec((_D, _D), full),
            pl.BlockSpec((_D, _D), full),
        ],
        out_specs=[
            pl.BlockSpec((_BLK, _D), blk),
            pl.BlockSpec((_BLK, _D), blk),
        ],
        out_shape=[
            jax.ShapeDtypeStruct((_N_NODES, _D), jnp.float32),
            jax.ShapeDtypeStruct((_N_NODES, _D), jnp.float32),
        ],
    )(x, agg, w1, w2)


def _tc_final(x, agg, gidx3d, inv_counts, fc_w, fc_b3d, reg_w, reg_b2d):
    blk = lambda i: (i, 0)
    return pl.pallas_call(
        _final_body,
        grid=(_NB,),
        in_specs=[
            pl.BlockSpec((_BLK, _D), blk),
            pl.BlockSpec((_BLK, _D), blk),
            pl.BlockSpec((1, 1, _BLK), lambda i: (i, 0, 0)),
            pl.BlockSpec((_N_GRAPHS, 1), lambda i: (0, 0)),
            pl.BlockSpec((2, _D, _D), lambda i: (0, 0, 0)),
            pl.BlockSpec((2, 1, _D), lambda i: (0, 0, 0)),
            pl.BlockSpec((1, _D), lambda i: (0, 0)),
            pl.BlockSpec((1, 1), lambda i: (0, 0)),
        ],
        out_specs=pl.BlockSpec((_N_GRAPHS, 1), lambda i: (0, 0)),
        out_shape=jax.ShapeDtypeStruct((_N_GRAPHS, 1), jnp.float32),
        scratch_shapes=[pltpu.VMEM((_N_GRAPHS, _D), jnp.float32)],
    )(x, agg, gidx3d, inv_counts, fc_w, fc_b3d, reg_w, reg_b2d)


def _sc_agg(m, zeros_acc, src_idx, tgt_idx, chunk_counts):
    mesh = plsc.VectorSubcoreMesh(core_axis_name="c", subcore_axis_name="s")

    @functools.partial(
        pl.kernel,
        out_type=jax.ShapeDtypeStruct((_NC, _NPAD, _D), jnp.float32),
        mesh=mesh,
        scratch_types=[
            pltpu.VMEM((_C, _CH), jnp.int32),
            pltpu.VMEM((_C, _CH), jnp.int32),
            pltpu.VMEM((_CH, _D), jnp.float32),
            pltpu.VMEM((_CH, _D), jnp.float32),
            pltpu.VMEM((_NS,), jnp.int32),
            pltpu.VMEM_SHARED((_NPAD, _D), jnp.float32),
            pltpu.SemaphoreType.DMA,
            pltpu.SemaphoreType.DMA,
        ],
    )
    def k(m_hbm, z_hbm, src_hbm, tgt_hbm, cnt_hbm, out_hbm, s_v, t_v,
          rows0_v, rows1_v, cnt_v, agg_sh, sem0, sem1):
        cid = lax.axis_index("c")
        sid = lax.axis_index("s")
        base = sid * _RPT

        pltpu.sync_copy(z_hbm.at[pl.ds(base, _RPT)],
                        agg_sh.at[pl.ds(base, _RPT)])

        pltpu.sync_copy(src_hbm.at[cid, sid], s_v)
        pltpu.sync_copy(tgt_hbm.at[cid, sid], t_v)
        pltpu.sync_copy(cnt_hbm.at[cid], cnt_v)
        n = cnt_v[pl.ds(sid, 1)][0]

        plsc.subcore_barrier()

        @pl.when(n > 0)
        def _():
            pltpu.async_copy(m_hbm.at[s_v.at[0]], rows0_v, sem0)

            def step(j, carry):
                jj = 2 * j
                pltpu.async_copy(m_hbm.at[s_v.at[jj + 1]], rows1_v, sem1)
                pltpu.make_async_copy(
                    m_hbm.at[s_v.at[jj]], rows0_v, sem0).wait()
                pltpu.sync_copy(rows0_v, agg_sh.at[t_v.at[jj]], add=True)
                nxt = jnp.minimum(jj + 2, n - 1)
                pltpu.async_copy(m_hbm.at[s_v.at[nxt]], rows0_v, sem0)
                pltpu.make_async_copy(
                    m_hbm.at[s_v.at[jj + 1]], rows1_v, sem1).wait()
                pltpu.sync_copy(rows1_v, agg_sh.at[t_v.at[jj + 1]], add=True)
                return carry
            lax.fori_loop(0, n // 2, step, 0)

            pltpu.make_async_copy(m_hbm.at[s_v.at[n - 1]], rows0_v,
                                  sem0).wait()

            @pl.when(n % 2 == 1)
            def _():
                pltpu.sync_copy(rows0_v, agg_sh.at[t_v.at[n - 1]], add=True)

        plsc.subcore_barrier()

        pltpu.sync_copy(agg_sh.at[pl.ds(base, _RPT)],
                        out_hbm.at[cid].at[pl.ds(base, _RPT)])

    return k(m, zeros_acc, src_idx, tgt_idx, chunk_counts)


def kernel(nodes, node_counts, edge_sources, edge_targets, graph_indices,
           emb_W, conv_W1, conv_W2, fc_W, fc_b, reg_W, reg_b):
    E = edge_sources.shape[0]
    pad = _EPAD - E
    eidx = jnp.arange(_EPAD, dtype=jnp.int32)
    ecnt = jnp.arange(E, dtype=jnp.int32)

    is1 = edge_targets >= _HN
    n0 = jnp.sum(~is1).astype(jnp.int32)
    c1 = jnp.cumsum(is1.astype(jnp.int32)).astype(jnp.int32)
    c0 = ecnt + 1 - c1
    pos0 = jnp.where(is1, n0 + c1 - 1, c0 - 1)
    pos1 = jnp.where(is1, c1 - 1, (E - n0) + c0 - 1)
    inv0 = jnp.zeros((E,), jnp.int32).at[pos0].set(ecnt, unique_indices=True)
    inv1 = jnp.zeros((E,), jnp.int32).at[pos1].set(ecnt, unique_indices=True)

    pad_src = eidx[:pad] % _N_NODES
    pad_tgt = jnp.full((pad,), _N_NODES, jnp.int32)
    src0 = jnp.concatenate([edge_sources[inv0], pad_src])
    src1 = jnp.concatenate([edge_sources[inv1], pad_src])
    t0 = jnp.concatenate([edge_targets[inv0], pad_tgt])
    t1 = jnp.concatenate([edge_targets[inv1], pad_tgt])
    tgt0 = jnp.where(t0 < _HN, t0, _HN + (eidx % (_NPAD - _HN)))
    tgt1 = jnp.where((t1 >= _HN) & (t1 < _N_NODES), t1 - _HN,
                     _HN1 + (eidx % (_NPAD - _HN1)))

    def _dist(a):
        return a.reshape(_C, _NS, _CH).transpose(1, 0, 2)
    src_p = jnp.stack([_dist(src0), _dist(src1)])
    tgt_p = jnp.stack([_dist(tgt0), _dist(tgt1)])

    sgrid = jnp.arange(_NS, dtype=jnp.int32)
    v0 = (n0 + _CH - 1) // _CH
    v1 = ((E - n0) + _CH - 1) // _CH
    chunk_counts = jnp.stack(
        [jnp.maximum(0, (v0 - sgrid + _NS - 1) // _NS).astype(jnp.int32),
         jnp.maximum(0, (v1 - sgrid + _NS - 1) // _NS).astype(jnp.int32)])
    zeros_acc = jnp.zeros((_NPAD, _D), jnp.float32)

    x, m = _tc_embed_msg(nodes, emb_W, conv_W1[0], conv_W2[0])
    for i in range(_N_CONV):
        partials = _sc_agg(m, zeros_acc, src_p, tgt_p, chunk_counts)
        agg = jnp.concatenate(
            [partials[0, :_HN], partials[1, :_N_NODES - _HN]])
        if i + 1 < _N_CONV:
            x, m = _tc_update_msg(x, agg, conv_W1[i + 1], conv_W2[i + 1])

    gidx3d = graph_indices.reshape(_NB, 1, _BLK)
    inv_counts = (1.0 / node_counts).reshape(_N_GRAPHS, 1)
    out2d = _tc_final(x, agg, gidx3d, inv_counts, fc_W,
                      fc_b.reshape(2, 1, _D), reg_W, reg_b.reshape(1, 1))
    return out2d[:, 0]
